# baseline (speedup 1.0000x reference)
import jax
import jax.numpy as jnp
from jax import lax
from jax.experimental import pallas as pl
from jax.experimental.pallas import tpu as pltpu, tpu_sc as plsc

B = 4096
L = 200
D = 32
OUT_D = 3 * D + 1

GOODS_V = 1000000
SHOP_V = 100000
CATE_V = 1000

NUM_WORKERS = 32
TL = 8
TB = 128
BLK = TL * TB
HB = BLK // 2
NBLK = L // TL
GRPS = HB // 16


def _transpose_half(rb, planebuf, half, lane16):
    @pl.loop(0, GRPS)
    def _grp(k):
        rvec = lane16 + k * 16
        r = half * 4 + k // 8
        c = lax.rem(k, 8) * 16
        for d in range(D):
            vals = plsc.load_gather(rb, [rvec, jnp.full((16,), d, jnp.int32)])
            planebuf[d, r, pl.ds(c, 16)] = vals


def _body(goods_t, shop_t, cate_t, prices_t, gids, sids, cids, out,
          idg, ids, idc, pbuf, rb0, rb1, planebuf, catebuf,
          isem, gsem, wsem):
    wid = lax.axis_index("s") * 2 + lax.axis_index("c")
    lane16 = lax.iota(jnp.int32, 16)

    pltpu.sync_copy(cate_t, catebuf)

    def id_copies(n, sl):
        return [pltpu.make_async_copy(gids.at[n, wid], idg.at[sl], isem),
                pltpu.make_async_copy(sids.at[n, wid], ids.at[sl], isem),
                pltpu.make_async_copy(cids.at[n, wid], idc.at[sl], isem),
                pltpu.make_async_copy(
                    prices_t.at[pl.ds(n * TL, TL), pl.ds(wid * TB, TB)],
                    pbuf.at[sl], isem)]

    def gather_copies(table, idbuf, sl, half, rb):
        cps = []
        for j in range(TL // 2):
            r = half * (TL // 2) + j
            cps.append(pltpu.make_async_copy(
                table.at[idbuf.at[sl, r]],
                rb.at[pl.ds(j * TB, TB)], gsem))
        return cps

    def plane_writes(toff, n):
        return [pltpu.make_async_copy(planebuf.at[d],
                                      out.at[toff + d, n, wid], wsem)
                for d in range(D)]

    def price_write(n, sl):
        return pltpu.make_async_copy(pbuf.at[sl], out.at[3 * D, n, wid], wsem)

    def cate_planes(sl):
        @pl.loop(0, GRPS * 2)
        def _grp(k):
            r = k // 8
            c = lax.rem(k, 8) * 16
            ids16 = idc[sl, r, pl.ds(c, 16)]
            for d in range(D):
                vals = plsc.load_gather(
                    catebuf, [ids16, jnp.full((16,), d, jnp.int32)])
                planebuf[d, r, pl.ds(c, 16)] = vals

    for cc in id_copies(0, 0):
        cc.start()

    @pl.loop(0, NBLK)
    def _blk(n):
        sl = lax.rem(n, 2)
        for cc in id_copies(n, sl):
            cc.wait()

        for cc in gather_copies(goods_t, idg, sl, 0, rb0):
            cc.start()
        for cc in gather_copies(goods_t, idg, sl, 1, rb1):
            cc.start()

        @pl.when(n > 0)
        def _drain_prev():
            for cc in plane_writes(2 * D, n - 1):
                cc.wait()
            price_write(n - 1, 1 - sl).wait()

        @pl.when(n < NBLK - 1)
        def _prefetch():
            for cc in id_copies(n + 1, 1 - sl):
                cc.start()

        for cc in gather_copies(goods_t, idg, sl, 0, rb0):
            cc.wait()
        _transpose_half(rb0, planebuf, 0, lane16)
        for cc in gather_copies(goods_t, idg, sl, 1, rb1):
            cc.wait()
        for cc in gather_copies(shop_t, ids, sl, 0, rb0):
            cc.start()
        _transpose_half(rb1, planebuf, 1, lane16)
        for cc in plane_writes(0, n):
            cc.start()

        for cc in gather_copies(shop_t, ids, sl, 0, rb0):
            cc.wait()
        for cc in gather_copies(shop_t, ids, sl, 1, rb1):
            cc.start()
        for cc in plane_writes(0, n):
            cc.wait()
        _transpose_half(rb0, planebuf, 0, lane16)
        for cc in gather_copies(shop_t, ids, sl, 1, rb1):
            cc.wait()
        _transpose_half(rb1, planebuf, 1, lane16)
        for cc in plane_writes(D, n):
            cc.start()

        for cc in plane_writes(D, n):
            cc.wait()
        cate_planes(sl)
        for cc in plane_writes(2 * D, n):
            cc.start()
        price_write(n, sl).start()

    for cc in plane_writes(2 * D, NBLK - 1):
        cc.wait()
    price_write(NBLK - 1, lax.rem(NBLK - 1, 2)).wait()


@jax.jit
def _sc_lookup(goods_t, shop_t, cate_t, prices_t, gids4d, sids4d, cids4d):
    mesh = plsc.VectorSubcoreMesh(core_axis_name="c", subcore_axis_name="s")
    return pl.kernel(
        _body,
        out_type=jax.ShapeDtypeStruct((OUT_D, NBLK, NUM_WORKERS, TL, TB),
                                      jnp.float32),
        mesh=mesh,
        compiler_params=pltpu.CompilerParams(use_tc_tiling_on_sc=False,
                                            needs_layout_passes=False),
        scratch_types=[
            pltpu.VMEM((2, TL, TB), jnp.int32),
            pltpu.VMEM((2, TL, TB), jnp.int32),
            pltpu.VMEM((2, TL, TB), jnp.int32),
            pltpu.VMEM((2, TL, TB), jnp.float32),
            pltpu.VMEM((HB, D), jnp.float32),
            pltpu.VMEM((HB, D), jnp.float32),
            pltpu.VMEM((D, TL, TB), jnp.float32),
            pltpu.VMEM((CATE_V, D), jnp.float32),
            pltpu.SemaphoreType.DMA,
            pltpu.SemaphoreType.DMA,
            pltpu.SemaphoreType.DMA,
        ],
    )(goods_t, shop_t, cate_t, prices_t, gids4d, sids4d, cids4d)


def _tile_view(ids):
    return (ids.T.reshape(NBLK, TL, NUM_WORKERS, TB)
            .transpose(0, 2, 1, 3).astype(jnp.int32))


def kernel(goods_table, shop_table, cate_table, goods_prices,
           goods_ids, shop_ids, cate_ids):
    gids4d = _tile_view(goods_ids)
    sids4d = _tile_view(shop_ids)
    cids4d = _tile_view(cate_ids)
    prices_t = goods_prices.transpose(1, 0, 2).reshape(L, B)
    out5d = _sc_lookup(goods_table, shop_table, cate_table,
                       prices_t, gids4d, sids4d, cids4d)
    return out5d.transpose(2, 4, 1, 3, 0).reshape(B, L, OUT_D)


# speedup vs baseline: 1.3258x; 1.3258x over previous
import jax
import jax.numpy as jnp
from jax import lax
from jax.experimental import pallas as pl
from jax.experimental.pallas import tpu as pltpu, tpu_sc as plsc

B = 4096
L = 200
D = 32
OUT_D = 3 * D + 1

GOODS_V = 1000000
SHOP_V = 100000
CATE_V = 1000

NUM_WORKERS = 32
TL = 8
TB = 128
BLK = TL * TB
HB = BLK // 2
NBLK = L // TL
GRPS = HB // 16


def _transpose_half(rb, planebuf, half, lane16):
    @plsc.parallel_loop(0, GRPS, unroll=4)
    def _grp(k):
        rvec = lane16 + k * 16
        r = half * 4 + k // 8
        c = lax.rem(k, 8) * 16
        for d in range(D):
            vals = plsc.load_gather(rb, [rvec, jnp.full((16,), d, jnp.int32)])
            planebuf[d, r, pl.ds(c, 16)] = vals


def _body(goods_t, shop_t, cate_t, prices_t, gids, sids, cids, out,
          idg, ids, idc, pbuf, rb0, rb1, planebuf, catebuf,
          isem, gsem, wsem):
    wid = lax.axis_index("s") * 2 + lax.axis_index("c")
    lane16 = lax.iota(jnp.int32, 16)

    pltpu.sync_copy(cate_t, catebuf)

    def id_copies(n, sl):
        return [pltpu.make_async_copy(gids.at[n, wid], idg.at[sl], isem),
                pltpu.make_async_copy(sids.at[n, wid], ids.at[sl], isem),
                pltpu.make_async_copy(cids.at[n, wid], idc.at[sl], isem),
                pltpu.make_async_copy(
                    prices_t.at[pl.ds(n * TL, TL), pl.ds(wid * TB, TB)],
                    pbuf.at[sl], isem)]

    def gather_copies(table, idbuf, sl, half, rb):
        cps = []
        for j in range(TL // 2):
            r = half * (TL // 2) + j
            cps.append(pltpu.make_async_copy(
                table.at[idbuf.at[sl, r]],
                rb.at[pl.ds(j * TB, TB)], gsem))
        return cps

    def plane_writes(toff, n):
        return [pltpu.make_async_copy(planebuf.at[d],
                                      out.at[toff + d, n, wid], wsem)
                for d in range(D)]

    def price_write(n, sl):
        return pltpu.make_async_copy(pbuf.at[sl], out.at[3 * D, n, wid], wsem)

    def cate_planes(sl):
        @plsc.parallel_loop(0, GRPS * 2, unroll=4)
        def _grp(k):
            r = k // 8
            c = lax.rem(k, 8) * 16
            ids16 = idc[sl, r, pl.ds(c, 16)]
            for d in range(D):
                vals = plsc.load_gather(
                    catebuf, [ids16, jnp.full((16,), d, jnp.int32)])
                planebuf[d, r, pl.ds(c, 16)] = vals

    for cc in id_copies(0, 0):
        cc.start()

    @pl.loop(0, NBLK)
    def _blk(n):
        sl = lax.rem(n, 2)
        for cc in id_copies(n, sl):
            cc.wait()

        for cc in gather_copies(goods_t, idg, sl, 0, rb0):
            cc.start()
        for cc in gather_copies(goods_t, idg, sl, 1, rb1):
            cc.start()

        @pl.when(n > 0)
        def _drain_prev():
            for cc in plane_writes(2 * D, n - 1):
                cc.wait()
            price_write(n - 1, 1 - sl).wait()

        @pl.when(n < NBLK - 1)
        def _prefetch():
            for cc in id_copies(n + 1, 1 - sl):
                cc.start()

        for cc in gather_copies(goods_t, idg, sl, 0, rb0):
            cc.wait()
        _transpose_half(rb0, planebuf, 0, lane16)
        for cc in gather_copies(goods_t, idg, sl, 1, rb1):
            cc.wait()
        for cc in gather_copies(shop_t, ids, sl, 0, rb0):
            cc.start()
        _transpose_half(rb1, planebuf, 1, lane16)
        for cc in plane_writes(0, n):
            cc.start()

        for cc in gather_copies(shop_t, ids, sl, 0, rb0):
            cc.wait()
        for cc in gather_copies(shop_t, ids, sl, 1, rb1):
            cc.start()
        for cc in plane_writes(0, n):
            cc.wait()
        _transpose_half(rb0, planebuf, 0, lane16)
        for cc in gather_copies(shop_t, ids, sl, 1, rb1):
            cc.wait()
        _transpose_half(rb1, planebuf, 1, lane16)
        for cc in plane_writes(D, n):
            cc.start()

        for cc in plane_writes(D, n):
            cc.wait()
        cate_planes(sl)
        for cc in plane_writes(2 * D, n):
            cc.start()
        price_write(n, sl).start()

    for cc in plane_writes(2 * D, NBLK - 1):
        cc.wait()
    price_write(NBLK - 1, lax.rem(NBLK - 1, 2)).wait()


@jax.jit
def _sc_lookup(goods_t, shop_t, cate_t, prices_t, gids4d, sids4d, cids4d):
    mesh = plsc.VectorSubcoreMesh(core_axis_name="c", subcore_axis_name="s")
    return pl.kernel(
        _body,
        out_type=jax.ShapeDtypeStruct((OUT_D, NBLK, NUM_WORKERS, TL, TB),
                                      jnp.float32),
        mesh=mesh,
        compiler_params=pltpu.CompilerParams(use_tc_tiling_on_sc=False,
                                            needs_layout_passes=False),
        scratch_types=[
            pltpu.VMEM((2, TL, TB), jnp.int32),
            pltpu.VMEM((2, TL, TB), jnp.int32),
            pltpu.VMEM((2, TL, TB), jnp.int32),
            pltpu.VMEM((2, TL, TB), jnp.float32),
            pltpu.VMEM((HB, D), jnp.float32),
            pltpu.VMEM((HB, D), jnp.float32),
            pltpu.VMEM((D, TL, TB), jnp.float32),
            pltpu.VMEM((CATE_V, D), jnp.float32),
            pltpu.SemaphoreType.DMA,
            pltpu.SemaphoreType.DMA,
            pltpu.SemaphoreType.DMA,
        ],
    )(goods_t, shop_t, cate_t, prices_t, gids4d, sids4d, cids4d)


def _tile_view(ids):
    return (ids.T.reshape(NBLK, TL, NUM_WORKERS, TB)
            .transpose(0, 2, 1, 3).astype(jnp.int32))


def kernel(goods_table, shop_table, cate_table, goods_prices,
           goods_ids, shop_ids, cate_ids):
    gids4d = _tile_view(goods_ids)
    sids4d = _tile_view(shop_ids)
    cids4d = _tile_view(cate_ids)
    prices_t = goods_prices.transpose(1, 0, 2).reshape(L, B)
    out5d = _sc_lookup(goods_table, shop_table, cate_table,
                       prices_t, gids4d, sids4d, cids4d)
    return out5d.transpose(2, 4, 1, 3, 0).reshape(B, L, OUT_D)


# speedup vs baseline: 1.3866x; 1.0458x over previous
import jax
import jax.numpy as jnp
from jax import lax
from jax.experimental import pallas as pl
from jax.experimental.pallas import tpu as pltpu, tpu_sc as plsc

B = 4096
L = 200
D = 32
OUT_D = 3 * D + 1

GOODS_V = 1000000
SHOP_V = 100000
CATE_V = 1000

NUM_WORKERS = 32
TL = 8
TB = 128
BLK = TL * TB
HB = BLK // 2
NBLK = L // TL
GRPS = HB // 16


def _transpose_half(rb, planebuf, half, lane16):
    @plsc.parallel_loop(0, GRPS, unroll=8)
    def _grp(k):
        rvec = lane16 + k * 16
        r = half * 4 + k // 8
        c = lax.rem(k, 8) * 16
        for d in range(D):
            vals = plsc.load_gather(rb, [rvec, jnp.full((16,), d, jnp.int32)])
            planebuf[d, r, pl.ds(c, 16)] = vals


def _body(goods_t, shop_t, cate_t, prices_t, gids, sids, cids, out,
          idg, ids, idc, pbuf, rb0, rb1, planebuf, catebuf,
          isem, gsem, wsem):
    wid = lax.axis_index("s") * 2 + lax.axis_index("c")
    lane16 = lax.iota(jnp.int32, 16)

    pltpu.sync_copy(cate_t, catebuf)

    def id_copies(n, sl):
        return [pltpu.make_async_copy(gids.at[n, wid], idg.at[sl], isem),
                pltpu.make_async_copy(sids.at[n, wid], ids.at[sl], isem),
                pltpu.make_async_copy(cids.at[n, wid], idc.at[sl], isem),
                pltpu.make_async_copy(
                    prices_t.at[pl.ds(n * TL, TL), pl.ds(wid * TB, TB)],
                    pbuf.at[sl], isem)]

    def gather_copies(table, idbuf, sl, half, rb):
        cps = []
        for j in range(TL // 2):
            r = half * (TL // 2) + j
            cps.append(pltpu.make_async_copy(
                table.at[idbuf.at[sl, r]],
                rb.at[pl.ds(j * TB, TB)], gsem))
        return cps

    def plane_writes(toff, n):
        return [pltpu.make_async_copy(planebuf.at[d],
                                      out.at[toff + d, n, wid], wsem)
                for d in range(D)]

    def price_write(n, sl):
        return pltpu.make_async_copy(pbuf.at[sl], out.at[3 * D, n, wid], wsem)

    def cate_planes(sl):
        @plsc.parallel_loop(0, GRPS * 2, unroll=8)
        def _grp(k):
            r = k // 8
            c = lax.rem(k, 8) * 16
            ids16 = idc[sl, r, pl.ds(c, 16)]
            for d in range(D):
                vals = plsc.load_gather(
                    catebuf, [ids16, jnp.full((16,), d, jnp.int32)])
                planebuf[d, r, pl.ds(c, 16)] = vals

    for cc in id_copies(0, 0):
        cc.start()

    @pl.loop(0, NBLK)
    def _blk(n):
        sl = lax.rem(n, 2)
        for cc in id_copies(n, sl):
            cc.wait()

        for cc in gather_copies(goods_t, idg, sl, 0, rb0):
            cc.start()
        for cc in gather_copies(goods_t, idg, sl, 1, rb1):
            cc.start()

        @pl.when(n > 0)
        def _drain_prev():
            for cc in plane_writes(2 * D, n - 1):
                cc.wait()
            price_write(n - 1, 1 - sl).wait()

        @pl.when(n < NBLK - 1)
        def _prefetch():
            for cc in id_copies(n + 1, 1 - sl):
                cc.start()

        for cc in gather_copies(goods_t, idg, sl, 0, rb0):
            cc.wait()
        _transpose_half(rb0, planebuf, 0, lane16)
        for cc in gather_copies(goods_t, idg, sl, 1, rb1):
            cc.wait()
        for cc in gather_copies(shop_t, ids, sl, 0, rb0):
            cc.start()
        _transpose_half(rb1, planebuf, 1, lane16)
        for cc in plane_writes(0, n):
            cc.start()

        for cc in gather_copies(shop_t, ids, sl, 0, rb0):
            cc.wait()
        for cc in gather_copies(shop_t, ids, sl, 1, rb1):
            cc.start()
        for cc in plane_writes(0, n):
            cc.wait()
        _transpose_half(rb0, planebuf, 0, lane16)
        for cc in gather_copies(shop_t, ids, sl, 1, rb1):
            cc.wait()
        _transpose_half(rb1, planebuf, 1, lane16)
        for cc in plane_writes(D, n):
            cc.start()

        for cc in plane_writes(D, n):
            cc.wait()
        cate_planes(sl)
        for cc in plane_writes(2 * D, n):
            cc.start()
        price_write(n, sl).start()

    for cc in plane_writes(2 * D, NBLK - 1):
        cc.wait()
    price_write(NBLK - 1, lax.rem(NBLK - 1, 2)).wait()


@jax.jit
def _sc_lookup(goods_t, shop_t, cate_t, prices_t, gids4d, sids4d, cids4d):
    mesh = plsc.VectorSubcoreMesh(core_axis_name="c", subcore_axis_name="s")
    return pl.kernel(
        _body,
        out_type=jax.ShapeDtypeStruct((OUT_D, NBLK, NUM_WORKERS, TL, TB),
                                      jnp.float32),
        mesh=mesh,
        compiler_params=pltpu.CompilerParams(use_tc_tiling_on_sc=False,
                                            needs_layout_passes=False),
        scratch_types=[
            pltpu.VMEM((2, TL, TB), jnp.int32),
            pltpu.VMEM((2, TL, TB), jnp.int32),
            pltpu.VMEM((2, TL, TB), jnp.int32),
            pltpu.VMEM((2, TL, TB), jnp.float32),
            pltpu.VMEM((HB, D), jnp.float32),
            pltpu.VMEM((HB, D), jnp.float32),
            pltpu.VMEM((D, TL, TB), jnp.float32),
            pltpu.VMEM((CATE_V, D), jnp.float32),
            pltpu.SemaphoreType.DMA,
            pltpu.SemaphoreType.DMA,
            pltpu.SemaphoreType.DMA,
        ],
    )(goods_t, shop_t, cate_t, prices_t, gids4d, sids4d, cids4d)


def _tile_view(ids):
    return (ids.T.reshape(NBLK, TL, NUM_WORKERS, TB)
            .transpose(0, 2, 1, 3).astype(jnp.int32))


def kernel(goods_table, shop_table, cate_table, goods_prices,
           goods_ids, shop_ids, cate_ids):
    gids4d = _tile_view(goods_ids)
    sids4d = _tile_view(shop_ids)
    cids4d = _tile_view(cate_ids)
    prices_t = goods_prices.transpose(1, 0, 2).reshape(L, B)
    out5d = _sc_lookup(goods_table, shop_table, cate_table,
                       prices_t, gids4d, sids4d, cids4d)
    return out5d.transpose(2, 4, 1, 3, 0).reshape(B, L, OUT_D)


# speedup vs baseline: 1.4852x; 1.0711x over previous
import jax
import jax.numpy as jnp
from jax import lax
from jax.experimental import pallas as pl
from jax.experimental.pallas import tpu as pltpu, tpu_sc as plsc

B = 4096
L = 200
D = 32
OUT_D = 3 * D + 1
N = B * L

NUM_WORKERS = 32
PER_W = N // NUM_WORKERS
CHUNK = 512
SUB = 128
NSUB = CHUNK // SUB
PAIR = 2 * CHUNK
RPP = PAIR // SUB
NPAIRS = PER_W // PAIR


def _body(goods_t, shop_t, cate_t, prices, gids, sids, cids, out,
          idxg, idxs, idxc, pbuf, pb20, pb21, g0, s0, c0, g1, s1, c1,
          isem, gsem, wsem):
    wid = lax.axis_index("s") * 2 + lax.axis_index("c")
    w_base = wid * PER_W
    lane = lax.iota(jnp.int32, 16)
    zero16 = jnp.zeros((16,), jnp.int32)

    def id_copies(p, sl):
        pair_base = w_base + p * PAIR
        rows = pl.ds(pair_base // SUB, RPP)
        return [pltpu.make_async_copy(gids.at[rows], idxg.at[sl], isem),
                pltpu.make_async_copy(sids.at[rows], idxs.at[sl], isem),
                pltpu.make_async_copy(cids.at[rows], idxc.at[sl], isem),
                pltpu.make_async_copy(prices.at[pl.ds(pair_base, PAIR)],
                                      pbuf.at[sl], isem)]

    def fill_price(ci, sl, pb2):
        for i in range(CHUNK // 16):
            vals = pbuf[sl, pl.ds(ci * CHUNK + i * 16, 16)]
            plsc.store_scatter(pb2, [lane + i * 16, zero16], vals)

    def gather_copies(ci, sl, gb, sb, cb):
        cps = []
        for j in range(NSUB):
            r = ci * NSUB + j
            rows = pl.ds(j * SUB, SUB)
            cps += [pltpu.make_async_copy(goods_t.at[idxg.at[sl, r]],
                                          gb.at[rows], gsem),
                    pltpu.make_async_copy(shop_t.at[idxs.at[sl, r]],
                                          sb.at[rows], gsem),
                    pltpu.make_async_copy(cate_t.at[idxc.at[sl, r]],
                                          cb.at[rows], gsem)]
        return cps

    def write_copies(p, ci, pb2, gb, sb, cb):
        base = w_base + p * PAIR + ci * CHUNK
        rows = pl.ds(base, CHUNK)
        return [pltpu.make_async_copy(gb, out.at[rows, pl.ds(0, D)], wsem),
                pltpu.make_async_copy(sb, out.at[rows, pl.ds(D, D)], wsem),
                pltpu.make_async_copy(cb, out.at[rows, pl.ds(2 * D, D)],
                                      wsem),
                pltpu.make_async_copy(pb2,
                                      out.at[rows, pl.ds(3 * D, 1)], wsem)]

    for c in id_copies(0, 0):
        c.start()

    @pl.loop(0, NPAIRS)
    def _pair(p):
        sl = lax.rem(p, 2)
        for c in id_copies(p, sl):
            c.wait()

        @pl.when(p > 0)
        def _drain0():
            for c in write_copies(p - 1, 0, pb20, g0, s0, c0):
                c.wait()
        for c in gather_copies(0, sl, g0, s0, c0):
            c.start()
        fill_price(0, sl, pb20)

        @pl.when(p > 0)
        def _drain1():
            for c in write_copies(p - 1, 1, pb21, g1, s1, c1):
                c.wait()
        for c in gather_copies(1, sl, g1, s1, c1):
            c.start()
        fill_price(1, sl, pb21)

        @pl.when(p < NPAIRS - 1)
        def _prefetch():
            for c in id_copies(p + 1, 1 - sl):
                c.start()

        for c in gather_copies(0, sl, g0, s0, c0):
            c.wait()
        for c in write_copies(p, 0, pb20, g0, s0, c0):
            c.start()
        for c in gather_copies(1, sl, g1, s1, c1):
            c.wait()
        for c in write_copies(p, 1, pb21, g1, s1, c1):
            c.start()

    last = NPAIRS - 1
    for c in write_copies(last, 0, pb20, g0, s0, c0):
        c.wait()
    for c in write_copies(last, 1, pb21, g1, s1, c1):
        c.wait()


@jax.jit
def _sc_lookup(goods_t, shop_t, cate_t, prices1d, gids2d, sids2d, cids2d):
    mesh = plsc.VectorSubcoreMesh(core_axis_name="c", subcore_axis_name="s")
    return pl.kernel(
        _body,
        out_type=jax.ShapeDtypeStruct((N, OUT_D), jnp.float32),
        mesh=mesh,
        compiler_params=pltpu.CompilerParams(use_tc_tiling_on_sc=False,
                                            needs_layout_passes=False),
        scratch_types=[
            pltpu.VMEM((2, RPP, SUB), jnp.int32),
            pltpu.VMEM((2, RPP, SUB), jnp.int32),
            pltpu.VMEM((2, RPP, SUB), jnp.int32),
            pltpu.VMEM((2, PAIR), jnp.float32),
            pltpu.VMEM((CHUNK, 1), jnp.float32),
            pltpu.VMEM((CHUNK, 1), jnp.float32),
            pltpu.VMEM((CHUNK, D), jnp.float32),
            pltpu.VMEM((CHUNK, D), jnp.float32),
            pltpu.VMEM((CHUNK, D), jnp.float32),
            pltpu.VMEM((CHUNK, D), jnp.float32),
            pltpu.VMEM((CHUNK, D), jnp.float32),
            pltpu.VMEM((CHUNK, D), jnp.float32),
            pltpu.SemaphoreType.DMA,
            pltpu.SemaphoreType.DMA,
            pltpu.SemaphoreType.DMA,
        ],
    )(goods_t, shop_t, cate_t, prices1d, gids2d, sids2d, cids2d)


def kernel(goods_table, shop_table, cate_table, goods_prices,
           goods_ids, shop_ids, cate_ids):
    gids2d = goods_ids.reshape(N // SUB, SUB).astype(jnp.int32)
    sids2d = shop_ids.reshape(N // SUB, SUB).astype(jnp.int32)
    cids2d = cate_ids.reshape(N // SUB, SUB).astype(jnp.int32)
    prices1d = goods_prices.reshape(N)
    out = _sc_lookup(goods_table, shop_table, cate_table,
                     prices1d, gids2d, sids2d, cids2d)
    return out.reshape(B, L, OUT_D)
